# baseline (speedup 1.0000x reference)
import functools

import jax
import jax.numpy as jnp
from jax import lax
from jax.experimental import pallas as pl
from jax.experimental.pallas import tpu as pltpu
from jax.experimental.pallas import tpu_sc as plsc

_DIM = 32
_BATCH = 16384
_FIELDS = 26

_R = 262144
_BR = 4096
_NBLK = _R // _BR

_B = _BATCH * _FIELDS
_NW = 32
_BPW = _B // _NW
_CHUNK = 1664
_NCHUNK = _BPW // _CHUNK


def _tc_relayout(table_t):
    def body(x0, x1, x2, x3, o):
        for k, xk in enumerate((x0, x1, x2, x3)):
            o[:, _DIM * k:_DIM * (k + 1)] = xk[...].T

    last_blk = (1_000_000 - 1) // _BR
    specs = [
        pl.BlockSpec(
            (_DIM, _BR), lambda i, k=k: (0, jnp.minimum(i + k * _NBLK, last_blk))
        )
        for k in range(4)
    ]
    return pl.pallas_call(
        body,
        grid=(_NBLK,),
        in_specs=specs,
        out_specs=pl.BlockSpec((_BR, 4 * _DIM), lambda i: (i, 0)),
        out_shape=jax.ShapeDtypeStruct((_R, 4 * _DIM), jnp.float32),
    )(table_t, table_t, table_t, table_t)


def _build_gather():
    mesh = plsc.VectorSubcoreMesh(core_axis_name="c", subcore_axis_name="s")

    _UNITS = _FIELDS * (_BATCH // 128)
    _UPW = _UNITS // _NW

    @functools.partial(
        pl.kernel,
        mesh=mesh,
        compiler_params=pltpu.CompilerParams(
            use_tc_tiling_on_sc=False, needs_layout_passes=False
        ),
        out_type=jax.ShapeDtypeStruct((_FIELDS, 4, 128, 8, 128), jnp.float32),
        scratch_types=[
            pltpu.VMEM((128,), jnp.int32),
            pltpu.VMEM((128, _DIM), jnp.float32),
            pltpu.VMEM((4, 8, 128), jnp.float32),
            pltpu.SemaphoreType.DMA,
            pltpu.SemaphoreType.DMA,
        ],
    )
    def gather_kernel(idx_hbm, table_hbm, out_hbm, idxb, stage, otile, gsem, osem):
        wid = lax.axis_index("s") * 2 + lax.axis_index("c")
        u0 = wid * _UPW

        def unit(t, carry):
            u = u0 + t
            f = u >> 7
            bt = u & 127
            pltpu.sync_copy(idx_hbm.at[pl.ds(f * _BATCH + bt * 128, 128)], idxb)
            pltpu.async_copy(table_hbm.at[idxb], stage, gsem).wait()

            def group(g, c2):
                d = g >> 3
                bb = (g & 7) * 16
                ib = bb + lax.iota(jnp.int32, 16)
                ic = jnp.zeros((16,), jnp.int32) + d
                vals = plsc.load_gather(stage, [ib, ic])
                otile[d >> 3, d & 7, pl.ds(bb, 16)] = vals
                return c2

            lax.fori_loop(0, 256, group, 0)
            pltpu.async_copy(otile, out_hbm.at[f, :, bt, :, :], osem).wait()
            return carry

        lax.fori_loop(0, _UPW, unit, 0)

    return gather_kernel


_GATHER = _build_gather()


def kernel(indices, table):
    s128 = _tc_relayout(table.T)
    s1m = s128.reshape(4 * _R, _DIM)
    flat = indices.T.reshape(-1)
    sidx = ((flat & (_R - 1)) << 2) | (flat >> 18)
    out5 = _GATHER(sidx, s1m)
    return out5.transpose(2, 4, 0, 1, 3).reshape(_BATCH, _FIELDS, _DIM)


# speedup vs baseline: 1.4621x; 1.4621x over previous
import functools

import jax
import jax.numpy as jnp
from jax import lax
from jax.experimental import pallas as pl
from jax.experimental.pallas import tpu as pltpu
from jax.experimental.pallas import tpu_sc as plsc

_DIM = 32
_BATCH = 16384
_FIELDS = 26

_R = 262144
_BR = 4096
_NBLK = _R // _BR

_B = _BATCH * _FIELDS
_NW = 32
_BPW = _B // _NW
_CHUNK = 1664
_NCHUNK = _BPW // _CHUNK


def _tc_relayout(table_t):
    def body(x0, x1, x2, x3, o):
        for k, xk in enumerate((x0, x1, x2, x3)):
            o[:, _DIM * k:_DIM * (k + 1)] = xk[...].T

    last_blk = (1_000_000 - 1) // _BR
    specs = [
        pl.BlockSpec(
            (_DIM, _BR), lambda i, k=k: (0, jnp.minimum(i + k * _NBLK, last_blk))
        )
        for k in range(4)
    ]
    return pl.pallas_call(
        body,
        grid=(_NBLK,),
        in_specs=specs,
        out_specs=pl.BlockSpec((_BR, 4 * _DIM), lambda i: (i, 0)),
        out_shape=jax.ShapeDtypeStruct((_R, 4 * _DIM), jnp.float32),
    )(table_t, table_t, table_t, table_t)


def _build_gather():
    mesh = plsc.VectorSubcoreMesh(core_axis_name="c", subcore_axis_name="s")

    @functools.partial(
        pl.kernel,
        mesh=mesh,
        compiler_params=pltpu.CompilerParams(use_tc_tiling_on_sc=False),
        out_type=jax.ShapeDtypeStruct((_B, _DIM), jnp.float32),
        scratch_types=[
            pltpu.VMEM((_BPW,), jnp.int32),
            pltpu.VMEM((_CHUNK, _DIM), jnp.float32),
            pltpu.VMEM((_CHUNK, _DIM), jnp.float32),
            pltpu.SemaphoreType.DMA,
            pltpu.SemaphoreType.DMA,
            pltpu.SemaphoreType.DMA,
            pltpu.SemaphoreType.DMA,
        ],
    )
    def gather_kernel(idx_hbm, table_hbm, out_hbm, idx_v, r0, r1, g0, g1, s0, s1):
        wid = lax.axis_index("s") * 2 + lax.axis_index("c")
        base = wid * _BPW
        pltpu.sync_copy(idx_hbm.at[pl.ds(base, _BPW)], idx_v)
        bufs = (r0, r1)
        gsem = (g0, g1)
        ssem = (s0, s1)
        gathers = [None, None]
        stores = [None, None]
        for i in range(_NCHUNK):
            b = i & 1
            if stores[b] is not None:
                stores[b].wait()
            idx_slice = idx_v.at[pl.ds(i * _CHUNK, _CHUNK)]
            gathers[b] = pltpu.async_copy(table_hbm.at[idx_slice], bufs[b], gsem[b])
            if i > 0:
                pb = (i - 1) & 1
                gathers[pb].wait()
                stores[pb] = pltpu.async_copy(
                    bufs[pb], out_hbm.at[pl.ds(base + (i - 1) * _CHUNK, _CHUNK)], ssem[pb]
                )
        last = (_NCHUNK - 1) & 1
        gathers[last].wait()
        stores[last] = pltpu.async_copy(
            bufs[last], out_hbm.at[pl.ds(base + (_NCHUNK - 1) * _CHUNK, _CHUNK)], ssem[last]
        )
        stores[1 - last].wait()
        stores[last].wait()

    return gather_kernel


_GATHER = _build_gather()


def kernel(indices, table):
    s128 = _tc_relayout(table.T)
    s1m = s128.reshape(4 * _R, _DIM)
    flat = indices.T.reshape(-1)
    sidx = ((flat & (_R - 1)) << 2) | (flat >> 18)
    out = _GATHER(sidx, s1m)
    return out.reshape(_FIELDS, _BATCH, _DIM).transpose(1, 0, 2)
